# Initial kernel scaffold; baseline (speedup 1.0000x reference)
import jax
import jax.numpy as jnp
from jax.experimental import pallas as pl


def kernel(x, Wg, bg, W1, b1, W2, b2):
    raise NotImplementedError("write your pallas kernel here")


# speedup vs baseline: 2.8900x; 2.8900x over previous
import functools

import jax
import jax.numpy as jnp
from jax.experimental import pallas as pl

B = 768
S = 8
D = 768
E = 8
K = 2
HID = 1536
EPS = 1e-6
CAP = 3.0


def _gate_kernel(x_ref, wgt_ref, bg_ref, scale_ref):
    x2 = x_ref[...].reshape(B * S, D)
    logits = jnp.dot(x2, wgt_ref[...], preferred_element_type=jnp.float32)
    logits = logits + bg_ref[...]
    m = jnp.max(logits, axis=-1, keepdims=True)
    ex = jnp.exp(logits - m)
    p = ex / jnp.sum(ex, axis=-1, keepdims=True)
    e_iota = jax.lax.broadcasted_iota(jnp.int32, p.shape, 1)
    top1i = jnp.argmax(p, axis=-1).astype(jnp.int32)
    pm = jnp.where(e_iota == top1i[:, None], -jnp.inf, p)
    top2i = jnp.argmax(pm, axis=-1).astype(jnp.int32)
    t1 = top1i.reshape(B, S)
    t2 = top2i.reshape(B, S)
    si = jax.lax.broadcasted_iota(jnp.int32, (B, S, S), 2)
    mask1 = jnp.any(t1[:, :, None] == si, axis=1)
    mask2 = jnp.any(t2[:, :, None] == si, axis=1)
    pr = p.reshape(B, S, E)
    m0 = pr[:, :, 0] * mask1.astype(jnp.float32)
    m1 = pr[:, :, 1] * mask2.astype(jnp.float32)
    d0 = jnp.sum(m0, axis=0, keepdims=True) + EPS
    d1 = jnp.sum(m1, axis=0, keepdims=True) + EPS
    scale_ref[...] = (m0 / d0 + m1 / d1) * CAP


def _erf(v):
    a1, a2, a3, a4, a5, pp = (
        0.254829592,
        -0.284496736,
        1.421413741,
        -1.453152027,
        1.061405429,
        0.3275911,
    )
    sgn = jnp.sign(v)
    av = jnp.abs(v)
    t = 1.0 / (1.0 + pp * av)
    y = 1.0 - (((((a5 * t + a4) * t) + a3) * t + a2) * t + a1) * t * jnp.exp(
        -av * av
    )
    return sgn * y


def _gelu_exact(z):
    return 0.5 * z * (1.0 + _erf(z * 0.7071067811865476))


def _ffn_kernel(x_ref, w1_ref, b1_ref, w2_ref, b2_ref, scale_ref, out_ref):
    s = pl.program_id(0)
    h = jnp.dot(w1_ref[...], x_ref[...], preferred_element_type=jnp.float32)
    h = _gelu_exact(h + b1_ref[...])
    o = jnp.dot(w2_ref[...], h, preferred_element_type=jnp.float32)
    o = o + b2_ref[...]
    onehot = (jax.lax.broadcasted_iota(jnp.int32, (1, S), 1) == s).astype(
        jnp.float32
    )
    scale_col = jnp.sum(scale_ref[...] * onehot, axis=1, keepdims=True)
    out_ref[...] = o * scale_col


@jax.jit
def kernel(x, Wg, bg, W1, b1, W2, b2):
    scale = pl.pallas_call(
        _gate_kernel,
        out_shape=jax.ShapeDtypeStruct((B, S), jnp.float32),
    )(x, Wg.T, bg.reshape(1, E))

    x2 = x.reshape(B, S * D)
    out2 = pl.pallas_call(
        _ffn_kernel,
        grid=(S,),
        in_specs=[
            pl.BlockSpec((B, D), lambda s: (0, s)),
            pl.BlockSpec((HID, B), lambda s: (0, 0)),
            pl.BlockSpec((HID, 1), lambda s: (0, 0)),
            pl.BlockSpec((D, HID), lambda s: (0, 0)),
            pl.BlockSpec((D, 1), lambda s: (0, 0)),
            pl.BlockSpec((B, S), lambda s: (0, 0)),
        ],
        out_specs=pl.BlockSpec((D, D), lambda s: (0, s)),
        out_shape=jax.ShapeDtypeStruct((D, S * D), jnp.float32),
    )(x2, W1, b1.reshape(HID, 1), W2, b2.reshape(D, 1), scale)

    return out2.reshape(D, S, D)


# speedup vs baseline: 4.2562x; 1.4728x over previous
import jax
import jax.numpy as jnp
from jax.experimental import pallas as pl

B = 768
S = 8
D = 768
E = 8
K = 2
HID = 1536
EPS = 1e-6
CAP = 3.0


def _gate_kernel(x_ref, wg_ref, bg_ref, scale_ref):
    x2 = x_ref[...].reshape(B * S, D)
    logits = jax.lax.dot_general(
        x2,
        wg_ref[...],
        dimension_numbers=(((1,), (1,)), ((), ())),
        preferred_element_type=jnp.float32,
    )
    logits = logits + bg_ref[...]
    m = jnp.max(logits, axis=-1, keepdims=True)
    ex = jnp.exp(logits - m)
    p = ex / jnp.sum(ex, axis=-1, keepdims=True)
    e_iota = jax.lax.broadcasted_iota(jnp.int32, p.shape, 1)
    top1i = jnp.argmax(p, axis=-1).astype(jnp.int32)
    pm = jnp.where(e_iota == top1i[:, None], -jnp.inf, p)
    top2i = jnp.argmax(pm, axis=-1).astype(jnp.int32)
    t1 = top1i.reshape(B, S)
    t2 = top2i.reshape(B, S)
    si = jax.lax.broadcasted_iota(jnp.int32, (B, S, S), 2)
    mask1 = jnp.any(t1[:, :, None] == si, axis=1)
    mask2 = jnp.any(t2[:, :, None] == si, axis=1)
    pr = p.reshape(B, S, E)
    m0 = pr[:, :, 0] * mask1.astype(jnp.float32)
    m1 = pr[:, :, 1] * mask2.astype(jnp.float32)
    d0 = jnp.sum(m0, axis=0, keepdims=True) + EPS
    d1 = jnp.sum(m1, axis=0, keepdims=True) + EPS
    scale_ref[...] = (m0 / d0 + m1 / d1) * CAP


def _erf(v):
    a1, a2, a3, a4, a5, pp = (
        0.254829592,
        -0.284496736,
        1.421413741,
        -1.453152027,
        1.061405429,
        0.3275911,
    )
    sgn = jnp.sign(v)
    av = jnp.abs(v)
    t = 1.0 / (1.0 + pp * av)
    y = 1.0 - (((((a5 * t + a4) * t) + a3) * t + a2) * t + a1) * t * jnp.exp(
        -av * av
    )
    return sgn * y


def _gelu_exact(z):
    return 0.5 * z * (1.0 + _erf(z * 0.7071067811865476))


LBLK = 128


def _ffn_kernel(x_ref, w1_ref, b1_ref, w2_ref, b2_ref, scale_ref, out_ref):
    xb = x_ref[...].reshape(B, S * LBLK)
    h = jnp.dot(w1_ref[...], xb, preferred_element_type=jnp.float32)
    h = _gelu_exact(h + b1_ref[...])
    o = jnp.dot(w2_ref[...], h, preferred_element_type=jnp.float32)
    o = o + b2_ref[...]
    o3 = o.reshape(D, S, LBLK) * scale_ref[...][:, :, None]
    out_ref[...] = o3


@jax.jit
def kernel(x, Wg, bg, W1, b1, W2, b2):
    scale = pl.pallas_call(
        _gate_kernel,
        out_shape=jax.ShapeDtypeStruct((B, S), jnp.float32),
    )(x, Wg, bg.reshape(1, E))

    out = pl.pallas_call(
        _ffn_kernel,
        grid=(D // LBLK,),
        in_specs=[
            pl.BlockSpec((B, S, LBLK), lambda j: (0, 0, j)),
            pl.BlockSpec((HID, B), lambda j: (0, 0)),
            pl.BlockSpec((HID, 1), lambda j: (0, 0)),
            pl.BlockSpec((D, HID), lambda j: (0, 0)),
            pl.BlockSpec((D, 1), lambda j: (0, 0)),
            pl.BlockSpec((B, S), lambda j: (0, 0)),
        ],
        out_specs=pl.BlockSpec((D, S, LBLK), lambda j: (0, 0, j)),
        out_shape=jax.ShapeDtypeStruct((D, S, D), jnp.float32),
    )(x, W1, b1.reshape(HID, 1), W2, b2.reshape(D, 1), scale)

    return out
